# Initial kernel scaffold; baseline (speedup 1.0000x reference)
import functools

import jax
import jax.numpy as jnp
from jax import lax
from jax.experimental import pallas as pl
from jax.experimental.pallas import tpu as pltpu
from jax.experimental.pallas import tpu_sc as plsc

NC = 2
NS = 16
NW = NC * NS
L = 16

C = 128
NCHUNK = 80
EPT = C * NCHUNK
E_PAD = NW * EPT
ACC_ROWS = 10240
CW = 16


def _sc_partials(x, src, dst, n_nodes, d_feat):
  rows_per_tile_acc = ACC_ROWS // NS
  rows_per_tile_out = n_nodes // NS

  mesh = plsc.VectorSubcoreMesh(core_axis_name="c", subcore_axis_name="s")

  @functools.partial(
      pl.kernel,
      mesh=mesh,
      out_type=(
          jax.ShapeDtypeStruct((NC, n_nodes, d_feat), jnp.float32),
          jax.ShapeDtypeStruct((NC, n_nodes, CW), jnp.float32),
      ),
      scratch_types=[
          pltpu.VMEM((NCHUNK, C), jnp.int32),
          pltpu.VMEM((NCHUNK, C), jnp.int32),
          pltpu.VMEM((C, d_feat), jnp.float32),
          pltpu.VMEM((C, d_feat), jnp.float32),
          pltpu.VMEM((C, CW), jnp.float32),
          pltpu.VMEM_SHARED((ACC_ROWS, d_feat), jnp.float32),
          pltpu.VMEM_SHARED((ACC_ROWS, CW), jnp.float32),
          pltpu.SemaphoreType.DMA,
          pltpu.SemaphoreType.DMA,
      ],
  )
  def k(x_hbm, src_hbm, dst_hbm, sums_hbm, cnts_hbm,
        src_v, dst_v, rows0, rows1, onesb, acc_sh, cnt_sh, sem0, sem1):
    cid = lax.axis_index("c")
    sid = lax.axis_index("s")
    wid = sid * NC + cid

    pltpu.sync_copy(src_hbm.at[wid], src_v)
    pltpu.sync_copy(dst_hbm.at[wid], dst_v)

    @pl.loop(0, C)
    def _(r):
      onesb[r, :] = jnp.zeros((L,), jnp.float32)
      for cc in range(d_feat // L):
        rows0[r, pl.ds(cc * L, L)] = jnp.zeros((L,), jnp.float32)

    abase = sid * rows_per_tile_acc
    for b in range(rows_per_tile_acc // C):
      pltpu.sync_copy(rows0, acc_sh.at[pl.ds(abase + b * C, C)])
      pltpu.sync_copy(onesb, cnt_sh.at[pl.ds(abase + b * C, C)])

    @pl.loop(0, C)
    def _(r):
      onesb[r, :] = jnp.ones((L,), jnp.float32)

    plsc.subcore_barrier()

    pltpu.async_copy(x_hbm.at[src_v.at[0]], rows0, sem0)
    pltpu.async_copy(x_hbm.at[src_v.at[1]], rows1, sem1)

    bufs = (rows0, rows1)
    sems = (sem0, sem1)

    @pl.loop(0, NCHUNK, step=2)
    def _(j):
      for b in range(2):
        cix = j + b
        buf = bufs[b]
        sem = sems[b]
        pltpu.make_async_copy(x_hbm.at[src_v.at[cix]], buf, sem).wait()
        d_idx = dst_v.at[cix]
        pltpu.sync_copy(buf, acc_sh.at[d_idx], add=True)
        pltpu.sync_copy(onesb, cnt_sh.at[d_idx], add=True)
        nxt = cix + 2

        @pl.when(nxt < NCHUNK)
        def _():
          pltpu.async_copy(x_hbm.at[src_v.at[nxt]], buf, sem)

    plsc.subcore_barrier()

    obase = sid * rows_per_tile_out
    pltpu.sync_copy(acc_sh.at[pl.ds(obase, rows_per_tile_out)],
                    sums_hbm.at[cid, pl.ds(obase, rows_per_tile_out)])
    pltpu.sync_copy(cnt_sh.at[pl.ds(obase, rows_per_tile_out)],
                    cnts_hbm.at[cid, pl.ds(obase, rows_per_tile_out)])

  return k(x, src, dst)


def _finalize(sums, cnts, n_nodes, d_feat):
  rb = 1000
  grid = n_nodes // rb

  def body(s_ref, c_ref, o_ref):
    s = s_ref[0] + s_ref[1]
    cnt = c_ref[0, :, 0:1] + c_ref[1, :, 0:1]
    o_ref[...] = s / jnp.maximum(cnt, 1.0)

  return pl.pallas_call(
      body,
      grid=(grid,),
      in_specs=[
          pl.BlockSpec((2, rb, d_feat), lambda i: (0, i, 0)),
          pl.BlockSpec((2, rb, CW), lambda i: (0, i, 0)),
      ],
      out_specs=pl.BlockSpec((rb, d_feat), lambda i: (i, 0)),
      out_shape=jax.ShapeDtypeStruct((n_nodes, d_feat), jnp.float32),
  )(sums, cnts)


@jax.jit
def kernel(x, edge_index):
  n_nodes, d_feat = x.shape
  n_edges = edge_index.shape[1]

  ei = edge_index.astype(jnp.int32)
  pad = E_PAD - n_edges
  src = jnp.concatenate([ei[0], jnp.zeros((pad,), jnp.int32)])
  dst = jnp.concatenate([ei[1], jnp.full((pad,), n_nodes, jnp.int32)])
  src = src.reshape(NW, NCHUNK, C)
  dst = dst.reshape(NW, NCHUNK, C)

  sums, cnts = _sc_partials(x, src, dst, n_nodes, d_feat)
  return _finalize(sums, cnts, n_nodes, d_feat)


# speedup vs baseline: 2.3209x; 2.3209x over previous
import functools

import jax
import jax.numpy as jnp
from jax import lax
from jax.experimental import pallas as pl
from jax.experimental.pallas import tpu as pltpu
from jax.experimental.pallas import tpu_sc as plsc

NC = 2
NS = 16
L = 16

C = 32
NPHASE = 8
NCHUNK = 80
EPT = C * NCHUNK * NPHASE
E_PAD = NS * EPT
ACC_ROWS = 10240


def _sc_aggregate(x, src, dst, n_nodes, d_feat):
  rows_per_tile = ACC_ROWS // NS

  mesh = plsc.VectorSubcoreMesh(
      core_axis_name="c", subcore_axis_name="s", num_cores=NC)

  @functools.partial(
      pl.kernel,
      mesh=mesh,
      out_type=jax.ShapeDtypeStruct((NC, ACC_ROWS, d_feat), jnp.float32),
      scratch_types=[
          pltpu.VMEM((NCHUNK, C), jnp.int32),
          pltpu.VMEM((NCHUNK, C), jnp.int32),
          pltpu.VMEM((C, d_feat), jnp.float32),
          pltpu.VMEM_SHARED((ACC_ROWS, d_feat), jnp.float32),
          pltpu.SemaphoreType.DMA,
      ],
  )
  def k(x_hbm, src_hbm, dst_hbm, out_hbm, src_v, dst_v, rows0, acc_sh, sem0):
    cid = lax.axis_index("c")
    sid = lax.axis_index("s")

    @pl.loop(0, C)
    def _(r):
      for cc in range(d_feat // L):
        rows0[r, pl.ds(cc * L, L)] = jnp.zeros((L,), jnp.float32)

    abase = sid * rows_per_tile
    for b in range(rows_per_tile // C):
      pltpu.sync_copy(rows0, acc_sh.at[pl.ds(abase + b * C, C)])

    @pl.when(cid == 1)
    def _():
      @pl.loop(0, C)
      def _(r):
        for cc in range(d_feat // L):
          rows0[r, pl.ds(cc * L, L)] = jnp.ones((L,), jnp.float32)

    plsc.subcore_barrier()

    for p in range(NPHASE):
      pltpu.sync_copy(dst_hbm.at[sid * NPHASE + p], dst_v)

      @pl.when(cid == 0)
      def _():
        pltpu.sync_copy(src_hbm.at[sid * NPHASE + p], src_v)

        @pl.loop(0, NCHUNK)
        def _(j):
          pltpu.async_copy(x_hbm.at[src_v.at[j]], rows0, sem0).wait()
          pltpu.sync_copy(rows0, acc_sh.at[dst_v.at[j]], add=True)

      @pl.when(cid == 1)
      def _():
        @pl.loop(0, NCHUNK)
        def _(j):
          pltpu.sync_copy(rows0, acc_sh.at[dst_v.at[j]], add=True)

    plsc.subcore_barrier()

    pltpu.sync_copy(acc_sh.at[pl.ds(abase, rows_per_tile)],
                    out_hbm.at[cid, pl.ds(abase, rows_per_tile)])

  return k(x, src, dst)


def _finalize(agg, n_nodes, d_feat):
  rb = 1000
  grid = n_nodes // rb

  def body(a_ref, o_ref):
    cnt = a_ref[1, :, 0:1]
    o_ref[...] = a_ref[0] / jnp.maximum(cnt, 1.0)

  return pl.pallas_call(
      body,
      grid=(grid,),
      in_specs=[pl.BlockSpec((NC, rb, d_feat), lambda i: (0, i, 0))],
      out_specs=pl.BlockSpec((rb, d_feat), lambda i: (i, 0)),
      out_shape=jax.ShapeDtypeStruct((n_nodes, d_feat), jnp.float32),
  )(agg)


@jax.jit
def kernel(x, edge_index):
  n_nodes, d_feat = x.shape
  n_edges = edge_index.shape[1]

  ei = edge_index.astype(jnp.int32)
  pad = E_PAD - n_edges
  src = jnp.concatenate([ei[0], jnp.zeros((pad,), jnp.int32)])
  dst = jnp.concatenate([ei[1], jnp.full((pad,), n_nodes, jnp.int32)])
  src = src.reshape(NS * NPHASE, NCHUNK, C)
  dst = dst.reshape(NS * NPHASE, NCHUNK, C)

  agg = _sc_aggregate(x, src, dst, n_nodes, d_feat)
  return _finalize(agg, n_nodes, d_feat)


# speedup vs baseline: 3.8928x; 1.6773x over previous
import functools

import jax
import jax.numpy as jnp
from jax import lax
from jax.experimental import pallas as pl
from jax.experimental.pallas import tpu as pltpu
from jax.experimental.pallas import tpu_sc as plsc

NC = 2
NS = 16
L = 16

C = 64
NPHASE = 4
NCHUNK = 80
EPT = C * NCHUNK * NPHASE
E_PAD = NS * EPT
ACC_ROWS = 10240


def _sc_aggregate(x, src, dst, n_nodes, d_feat):
  rows_per_tile = ACC_ROWS // NS

  mesh = plsc.VectorSubcoreMesh(
      core_axis_name="c", subcore_axis_name="s", num_cores=NC)

  @functools.partial(
      pl.kernel,
      mesh=mesh,
      out_type=jax.ShapeDtypeStruct((NC, ACC_ROWS, d_feat), jnp.float32),
      scratch_types=[
          pltpu.VMEM((NCHUNK, C), jnp.int32),
          pltpu.VMEM((NCHUNK, C), jnp.int32),
          pltpu.VMEM((C, d_feat), jnp.float32),
          pltpu.VMEM((C, d_feat), jnp.float32),
          pltpu.VMEM_SHARED((ACC_ROWS, d_feat), jnp.float32),
          pltpu.SemaphoreType.DMA,
          pltpu.SemaphoreType.DMA,
      ],
  )
  def k(x_hbm, src_hbm, dst_hbm, out_hbm,
        src_v, dst_v, rows0, rows1, acc_sh, sem0, sem1):
    cid = lax.axis_index("c")
    sid = lax.axis_index("s")

    @pl.loop(0, C)
    def _(r):
      for cc in range(d_feat // L):
        rows0[r, pl.ds(cc * L, L)] = jnp.zeros((L,), jnp.float32)

    abase = sid * rows_per_tile
    for b in range(rows_per_tile // C):
      pltpu.sync_copy(rows0, acc_sh.at[pl.ds(abase + b * C, C)])

    @pl.when(cid == 1)
    def _():
      @pl.loop(0, C)
      def _(r):
        for cc in range(d_feat // L):
          rows0[r, pl.ds(cc * L, L)] = jnp.ones((L,), jnp.float32)

    plsc.subcore_barrier()

    bufs = (rows0, rows1)
    sems = (sem0, sem1)

    for p in range(NPHASE):
      pltpu.sync_copy(dst_hbm.at[sid * NPHASE + p], dst_v)

      @pl.when(cid == 0)
      def _():
        pltpu.sync_copy(src_hbm.at[sid * NPHASE + p], src_v)

        pltpu.async_copy(x_hbm.at[src_v.at[0]], rows0, sem0)
        pltpu.async_copy(x_hbm.at[src_v.at[1]], rows1, sem1)

        @pl.loop(0, NCHUNK, step=2)
        def _(j):
          for b in range(2):
            cix = j + b
            pltpu.make_async_copy(
                x_hbm.at[src_v.at[cix]], bufs[b], sems[b]).wait()
            pltpu.sync_copy(bufs[b], acc_sh.at[dst_v.at[cix]], add=True)
            nxt = cix + 2

            @pl.when(nxt < NCHUNK)
            def _():
              pltpu.async_copy(x_hbm.at[src_v.at[nxt]], bufs[b], sems[b])

      @pl.when(cid == 1)
      def _():
        @pl.loop(0, NCHUNK)
        def _(j):
          pltpu.sync_copy(rows0, acc_sh.at[dst_v.at[j]], add=True)

    plsc.subcore_barrier()

    pltpu.sync_copy(acc_sh.at[pl.ds(abase, rows_per_tile)],
                    out_hbm.at[cid, pl.ds(abase, rows_per_tile)])

  return k(x, src, dst)


def _finalize(agg, n_nodes, d_feat):
  rb = 1000
  grid = n_nodes // rb

  def body(a_ref, o_ref):
    cnt = a_ref[1, :, 0:1]
    o_ref[...] = a_ref[0] / jnp.maximum(cnt, 1.0)

  return pl.pallas_call(
      body,
      grid=(grid,),
      in_specs=[pl.BlockSpec((NC, rb, d_feat), lambda i: (0, i, 0))],
      out_specs=pl.BlockSpec((rb, d_feat), lambda i: (i, 0)),
      out_shape=jax.ShapeDtypeStruct((n_nodes, d_feat), jnp.float32),
  )(agg)


@jax.jit
def kernel(x, edge_index):
  n_nodes, d_feat = x.shape
  n_edges = edge_index.shape[1]

  ei = edge_index.astype(jnp.int32)
  pad = E_PAD - n_edges
  src = jnp.concatenate([ei[0], jnp.zeros((pad,), jnp.int32)])
  dst = jnp.concatenate([ei[1], jnp.full((pad,), n_nodes, jnp.int32)])
  src = src.reshape(NS * NPHASE, NCHUNK, C)
  dst = dst.reshape(NS * NPHASE, NCHUNK, C)

  agg = _sc_aggregate(x, src, dst, n_nodes, d_feat)
  return _finalize(agg, n_nodes, d_feat)


# speedup vs baseline: 3.9129x; 1.0052x over previous
import functools

import jax
import jax.numpy as jnp
from jax import lax
from jax.experimental import pallas as pl
from jax.experimental.pallas import tpu as pltpu
from jax.experimental.pallas import tpu_sc as plsc

NC = 2
NS = 16
L = 16

C = 40
NPHASE = 8
NCHUNK = 64
EPT = C * NCHUNK * NPHASE
E_PAD = NS * EPT
ACC_ROWS = 10240


def _sc_aggregate(x, src, dst, n_nodes, d_feat):
  rows_per_tile = ACC_ROWS // NS

  mesh = plsc.VectorSubcoreMesh(
      core_axis_name="c", subcore_axis_name="s", num_cores=NC)

  @functools.partial(
      pl.kernel,
      mesh=mesh,
      out_type=jax.ShapeDtypeStruct((NC, ACC_ROWS, d_feat), jnp.float32),
      scratch_types=[
          pltpu.VMEM((NCHUNK, C), jnp.int32),
          pltpu.VMEM((NCHUNK, C), jnp.int32),
          pltpu.VMEM((C, d_feat), jnp.float32),
          pltpu.VMEM((C, d_feat), jnp.float32),
          pltpu.VMEM((C, d_feat), jnp.float32),
          pltpu.VMEM((C, d_feat), jnp.float32),
          pltpu.VMEM_SHARED((ACC_ROWS, d_feat), jnp.float32),
          [pltpu.SemaphoreType.DMA] * 4,
          [pltpu.SemaphoreType.DMA] * 4,
      ],
  )
  def k(x_hbm, src_hbm, dst_hbm, out_hbm,
        src_v, dst_v, rows0, rows1, rows2, rows3, acc_sh, gsems, ssems):
    cid = lax.axis_index("c")
    sid = lax.axis_index("s")

    @pl.loop(0, C)
    def _(r):
      for cc in range(d_feat // L):
        rows0[r, pl.ds(cc * L, L)] = jnp.zeros((L,), jnp.float32)

    abase = sid * rows_per_tile
    for b in range(rows_per_tile // C):
      pltpu.sync_copy(rows0, acc_sh.at[pl.ds(abase + b * C, C)])

    @pl.when(cid == 1)
    def _():
      @pl.loop(0, C)
      def _(r):
        for cc in range(d_feat // L):
          rows0[r, pl.ds(cc * L, L)] = jnp.ones((L,), jnp.float32)

    plsc.subcore_barrier()

    bufs = (rows0, rows1, rows2, rows3)

    def gather_start(cix, b):
      pltpu.async_copy(x_hbm.at[src_v.at[cix]], bufs[b], gsems[b])

    def gather_wait(cix, b):
      pltpu.make_async_copy(x_hbm.at[src_v.at[cix]], bufs[b], gsems[b]).wait()

    def scatter_start(cix, b):
      pltpu.async_copy(bufs[b], acc_sh.at[dst_v.at[cix]], ssems[b], add=True)

    def scatter_wait(cix, b):
      pltpu.make_async_copy(
          bufs[b], acc_sh.at[dst_v.at[cix]], ssems[b]).wait()

    for p in range(NPHASE):
      pltpu.sync_copy(dst_hbm.at[sid * NPHASE + p], dst_v)

      @pl.when(cid == 0)
      def _():
        pltpu.sync_copy(src_hbm.at[sid * NPHASE + p], src_v)

        gather_start(0, 0)
        gather_start(1, 1)

        @pl.loop(0, NCHUNK, step=4)
        def _(j):
          for b in range(4):
            cix = j + b
            b2 = (b + 2) % 4
            nxt = cix + 2

            @pl.when(nxt < NCHUNK)
            def _():
              @pl.when(cix >= 2)
              def _():
                scatter_wait(cix - 2, b2)

              gather_start(nxt, b2)

            gather_wait(cix, b)
            scatter_start(cix, b)

        for b in range(4):
          scatter_wait(NCHUNK - 4 + b, b)

      @pl.when(cid == 1)
      def _():
        @pl.loop(0, NCHUNK)
        def _(j):
          pltpu.sync_copy(rows0, acc_sh.at[dst_v.at[j]], add=True)

    plsc.subcore_barrier()

    pltpu.sync_copy(acc_sh.at[pl.ds(abase, rows_per_tile)],
                    out_hbm.at[cid, pl.ds(abase, rows_per_tile)])

  return k(x, src, dst)


def _finalize(agg, n_nodes, d_feat):
  rb = 1000
  grid = n_nodes // rb

  def body(a_ref, o_ref):
    cnt = a_ref[1, :, 0:1]
    o_ref[...] = a_ref[0] / jnp.maximum(cnt, 1.0)

  return pl.pallas_call(
      body,
      grid=(grid,),
      in_specs=[pl.BlockSpec((NC, rb, d_feat), lambda i: (0, i, 0))],
      out_specs=pl.BlockSpec((rb, d_feat), lambda i: (i, 0)),
      out_shape=jax.ShapeDtypeStruct((n_nodes, d_feat), jnp.float32),
  )(agg)


@jax.jit
def kernel(x, edge_index):
  n_nodes, d_feat = x.shape
  n_edges = edge_index.shape[1]

  ei = edge_index.astype(jnp.int32)
  pad = E_PAD - n_edges
  src = jnp.concatenate([ei[0], jnp.zeros((pad,), jnp.int32)])
  dst = jnp.concatenate([ei[1], jnp.full((pad,), n_nodes, jnp.int32)])
  src = src.reshape(NS * NPHASE, NCHUNK, C)
  dst = dst.reshape(NS * NPHASE, NCHUNK, C)

  agg = _sc_aggregate(x, src, dst, n_nodes, d_feat)
  return _finalize(agg, n_nodes, d_feat)
